# baseline (speedup 1.0000x reference)
import jax
import jax.numpy as jnp
import numpy as np
from jax.experimental import pallas as pl

V = 25
E = V * (V - 1)
H = 128
K = 4
C = 3
T = 20
N = 32
N_IN_ENC = C * T
TAU = 0.5
PRED_STEPS = 10
NSTEP = T - 1
B = 4
G = N // B

_BN_DIV = float(np.sqrt(1.0 + 1e-5))


def _elu(x):
    return jnp.where(x > 0, x, jnp.exp(x) - 1.0)


def _mlp(h, w1, b1, w2, b2, bn):
    h = _elu(jnp.dot(h, w1, preferred_element_type=jnp.float32) + b1)
    h = _elu(jnp.dot(h, w2, preferred_element_type=jnp.float32) + b2)
    return bn[0:1] * h / _BN_DIV + bn[1:2]


def _fwd_body(x_ref, s0_ref, s10_ref, u_ref, relcat_ref, relrt_ref,
              m1w1, m1b1, m1w2, m1b2, m1bn,
              m2w1, m2b1, m2w2, m2b2, m2bn,
              m3w1, m3b1, m3w2, m3b2, m3bn,
              m4w1, m4b1, m4w2, m4b2, m4bn,
              fow, fob,
              mg1w, mg1b, mg2wa, mg2ba, mg2wb, mg2bb, mg2wc, mg2bc,
              gwi, gbi, gwh,
              o1w, o1b, o2w, o2b, o3w, o3b,
              relcat_bf_ref, relrt_bf_ref, mg1w_bf, mg2wa_bf, mg2wb_bf, mg2wc_bf,
              preds_ref, prob_ref):
    BE = B * E
    relcat = relcat_ref[...]
    relrt = relrt_ref[...]

    x = x_ref[0]
    h = _mlp(x, m1w1[...], m1b1[...], m1w2[...], m1b2[...], m1bn[...])
    rs = jnp.dot(relcat, h, preferred_element_type=jnp.float32)
    eh = jnp.concatenate([rs[:BE], rs[BE:]], axis=1)
    h = _mlp(eh, m2w1[...], m2b1[...], m2w2[...], m2b2[...], m2bn[...])
    h_skip = h
    inc = jnp.dot(relrt, h, preferred_element_type=jnp.float32) / V
    h = _mlp(inc, m3w1[...], m3b1[...], m3w2[...], m3b2[...], m3bn[...])
    rs = jnp.dot(relcat, h, preferred_element_type=jnp.float32)
    eh = jnp.concatenate([rs[:BE], rs[BE:], h_skip], axis=1)
    h = _mlp(eh, m4w1[...], m4b1[...], m4w2[...], m4b2[...], m4bn[...])
    logits = jnp.dot(h, fow[...], preferred_element_type=jnp.float32) + fob[...]

    prob_ref[0] = jax.nn.softmax(logits, axis=-1)

    u = u_ref[0]
    g = -jnp.log(1e-10 - jnp.log(u + 1e-10))
    y = (logits + g) / TAU
    y_soft = jax.nn.softmax(y, axis=-1)
    idx = jnp.zeros((BE, 1), jnp.int32)
    best = y[:, 0:1]
    for j in range(1, K):
        c = y[:, j:j + 1] > best
        idx = jnp.where(c, j, idx)
        best = jnp.where(c, y[:, j:j + 1], best)
    lane = jax.lax.broadcasted_iota(jnp.int32, (BE, K), 1)
    hard = (lane == idx).astype(jnp.float32)
    edges = (hard - y_soft) + y_soft
    masks = [edges[:, k:k + 1] for k in range(1, K)]

    relcat_bf = relcat_bf_ref[...]
    relrt_bf = relrt_bf_ref[...]
    w1cat = mg1w_bf[...]
    b1cat = mg1b[...]
    w2s = (mg2wa_bf[...], mg2wb_bf[...], mg2wc_bf[...])
    b2s = (mg2ba[...], mg2bb[...], mg2bc[...])
    wi = gwi[...]; bi = gbi[...]; wh = gwh[...]
    seq0 = s0_ref[0]
    seq10 = s10_ref[0]

    hidden = jnp.zeros((B * V, H), jnp.float32)
    pred = None
    for s in range(NSTEP):
        ins = seq0 if s == 0 else (seq10 if s == PRED_STEPS else pred)
        rs = jnp.dot(relcat_bf, hidden.astype(jnp.bfloat16),
                     preferred_element_type=jnp.float32)
        pre = jnp.concatenate([rs[:BE], rs[BE:]], axis=1)
        m1 = jnp.tanh(jnp.dot(pre.astype(jnp.bfloat16), w1cat,
                              preferred_element_type=jnp.float32) + b1cat)
        msgs = jnp.zeros((BE, H), jnp.float32)
        for k in range(3):
            mk = jnp.tanh(jnp.dot(m1[:, k * H:(k + 1) * H].astype(jnp.bfloat16),
                                  w2s[k],
                                  preferred_element_type=jnp.float32) + b2s[k])
            msgs = msgs + (mk * masks[k]) / 3.0
        agg = jnp.dot(relrt_bf, msgs.astype(jnp.bfloat16),
                      preferred_element_type=jnp.float32) / 3.0
        gin = jnp.dot(ins, wi, preferred_element_type=jnp.float32) + bi
        gh = jnp.dot(agg, wh, preferred_element_type=jnp.float32)
        r = jax.nn.sigmoid(gin[:, :H] + gh[:, :H])
        i = jax.nn.sigmoid(gin[:, H:2 * H] + gh[:, H:2 * H])
        nn = jnp.tanh(gin[:, 2 * H:] + r * gh[:, 2 * H:])
        hidden = (1.0 - i) * nn + i * hidden
        p = jax.nn.relu(jnp.dot(hidden, o1w[...], preferred_element_type=jnp.float32) + o1b[...])
        p = jax.nn.relu(jnp.dot(p, o2w[...], preferred_element_type=jnp.float32) + o2b[...])
        p = jnp.dot(p, o3w[...], preferred_element_type=jnp.float32) + o3b[...]
        pred = ins + p
        preds_ref[0, :, s, :] = pred


def _row(b):
    return b.reshape(1, -1)


def kernel(data, params, rel_rec, rel_send):
    p = params
    x = jnp.transpose(data, (0, 3, 1, 2)).reshape(G, B * V, N_IN_ENC)
    seq = jnp.transpose(data, (0, 2, 3, 1))
    s0 = seq[:, 0].reshape(G, B * V, C)
    s10 = seq[:, PRED_STEPS].reshape(G, B * V, C)
    u = jax.random.uniform(jax.random.key(42), (N, E, K),
                           dtype=jnp.float32).reshape(G, B * E, K)
    eyeb = jnp.eye(B, dtype=jnp.float32)
    relcat = jnp.concatenate([jnp.kron(eyeb, rel_rec),
                              jnp.kron(eyeb, rel_send)], axis=0)
    relrt = jnp.kron(eyeb, jnp.transpose(rel_rec))

    weights = []
    for name in ('mlp1', 'mlp2', 'mlp3', 'mlp4'):
        weights += [p[name + '_fc1_w'], _row(p[name + '_fc1_b']),
                    p[name + '_fc2_w'], _row(p[name + '_fc2_b']),
                    jnp.stack([p[name + '_bn_g'], p[name + '_bn_b']])]
    weights += [p['fc_out_w'], _row(p['fc_out_b'])]
    weights += [jnp.concatenate([p['msg1_%d_w' % k] for k in (1, 2, 3)], axis=1),
                jnp.concatenate([_row(p['msg1_%d_b' % k]) for k in (1, 2, 3)], axis=1)]
    for k in (1, 2, 3):
        weights += [p['msg2_%d_w' % k], _row(p['msg2_%d_b' % k])]
    weights += [jnp.concatenate([p['input_r_w'], p['input_i_w'], p['input_n_w']], axis=1),
                jnp.concatenate([_row(p['input_r_b']), _row(p['input_i_b']),
                                 _row(p['input_n_b'])], axis=1),
                jnp.concatenate([p['hidden_r_w'], p['hidden_i_w'], p['hidden_n_w']], axis=1)]
    weights += [p['out_fc1_w'], _row(p['out_fc1_b']),
                p['out_fc2_w'], _row(p['out_fc2_b']),
                p['out_fc3_w'], _row(p['out_fc3_b'])]
    bf = jnp.bfloat16
    weights += [relcat.astype(bf), relrt.astype(bf),
                jnp.concatenate([p['msg1_%d_w' % k] for k in (1, 2, 3)],
                                axis=1).astype(bf)]
    weights += [p['msg2_%d_w' % k].astype(bf) for k in (1, 2, 3)]

    def wspec(w):
        nd = w.ndim
        return pl.BlockSpec(w.shape, lambda i, _nd=nd: (0,) * _nd)

    in_specs = [
        pl.BlockSpec((1, B * V, N_IN_ENC), lambda i: (i, 0, 0)),
        pl.BlockSpec((1, B * V, C), lambda i: (i, 0, 0)),
        pl.BlockSpec((1, B * V, C), lambda i: (i, 0, 0)),
        pl.BlockSpec((1, B * E, K), lambda i: (i, 0, 0)),
        pl.BlockSpec((2 * B * E, B * V), lambda i: (0, 0)),
        pl.BlockSpec((B * V, B * E), lambda i: (0, 0)),
    ] + [wspec(w) for w in weights]

    out_specs = [
        pl.BlockSpec((1, B * V, NSTEP, C), lambda i: (i, 0, 0, 0)),
        pl.BlockSpec((1, B * E, K), lambda i: (i, 0, 0)),
    ]
    out_shape = [
        jax.ShapeDtypeStruct((G, B * V, NSTEP, C), jnp.float32),
        jax.ShapeDtypeStruct((G, B * E, K), jnp.float32),
    ]

    preds, prob = pl.pallas_call(
        _fwd_body,
        grid=(G,),
        in_specs=in_specs,
        out_specs=out_specs,
        out_shape=out_shape,
    )(x, s0, s10, u, relcat, relrt, *weights)
    return preds.reshape(N, V, NSTEP, C), prob.reshape(N, E, K)


# speedup vs baseline: 1.0214x; 1.0214x over previous
import jax
import jax.numpy as jnp
import numpy as np
from jax.experimental import pallas as pl

V = 25
E = V * (V - 1)
H = 128
K = 4
C = 3
T = 20
N = 32
N_IN_ENC = C * T
TAU = 0.5
PRED_STEPS = 10
NSTEP = T - 1
B = 4
G = N // B

_BN_DIV = float(np.sqrt(1.0 + 1e-5))


def _elu(x):
    return jnp.where(x > 0, x, jnp.exp(x) - 1.0)


def _mlp(h, w1, b1, w2, b2, bn):
    h = _elu(jnp.dot(h, w1, preferred_element_type=jnp.float32) + b1)
    h = _elu(jnp.dot(h, w2, preferred_element_type=jnp.float32) + b2)
    return bn[0:1] * h / _BN_DIV + bn[1:2]


def _fwd_body(x_ref, s0_ref, s10_ref, u_ref, relcat_ref, relrt_ref,
              m1w1, m1b1, m1w2, m1b2, m1bn,
              m2w1, m2b1, m2w2, m2b2, m2bn,
              m3w1, m3b1, m3w2, m3b2, m3bn,
              m4w1, m4b1, m4w2, m4b2, m4bn,
              fow, fob,
              mg1b, mg2b,
              gwi, gbi, gwh,
              o1w, o1b, o2w, o2b, o3w, o3b,
              relrec_bf_ref, relsend_bf_ref, relrt_bf_ref,
              w1top_bf, w1bot_bf, w2cat_bf,
              preds_ref, prob_ref):
    bf = jnp.bfloat16
    BE = B * E
    relcat = relcat_ref[...]
    relrt = relrt_ref[...]

    x = x_ref[0]
    h = _mlp(x, m1w1[...], m1b1[...], m1w2[...], m1b2[...], m1bn[...])
    rs = jnp.dot(relcat, h, preferred_element_type=jnp.float32)
    eh = jnp.concatenate([rs[:BE], rs[BE:]], axis=1)
    h = _mlp(eh, m2w1[...], m2b1[...], m2w2[...], m2b2[...], m2bn[...])
    h_skip = h
    inc = jnp.dot(relrt, h, preferred_element_type=jnp.float32) / V
    h = _mlp(inc, m3w1[...], m3b1[...], m3w2[...], m3b2[...], m3bn[...])
    rs = jnp.dot(relcat, h, preferred_element_type=jnp.float32)
    eh = jnp.concatenate([rs[:BE], rs[BE:], h_skip], axis=1)
    h = _mlp(eh, m4w1[...], m4b1[...], m4w2[...], m4b2[...], m4bn[...])
    logits = jnp.dot(h, fow[...], preferred_element_type=jnp.float32) + fob[...]

    prob_ref[0] = jax.nn.softmax(logits, axis=-1)

    u = u_ref[0]
    g = -jnp.log(1e-10 - jnp.log(u + 1e-10))
    y = (logits + g) / TAU
    y_soft = jax.nn.softmax(y, axis=-1)
    idx = jnp.zeros((BE, 1), jnp.int32)
    best = y[:, 0:1]
    for j in range(1, K):
        c = y[:, j:j + 1] > best
        idx = jnp.where(c, j, idx)
        best = jnp.where(c, y[:, j:j + 1], best)
    lane = jax.lax.broadcasted_iota(jnp.int32, (BE, K), 1)
    hard = (lane == idx).astype(jnp.float32)
    edges = (hard - y_soft) + y_soft
    mk1 = edges[:, 1:2]
    mk2 = edges[:, 2:3]
    mk3 = edges[:, 3:4]

    relrec_bf = relrec_bf_ref[...]
    relsend_bf = relsend_bf_ref[...]
    relrt_bf = relrt_bf_ref[...]
    w1t = w1top_bf[...]
    w1b = w1bot_bf[...]
    w2c = w2cat_bf[...]
    b1c = mg1b[...]
    b2c = mg2b[...]
    wi = gwi[...]; bi = gbi[...]; wh = gwh[...]
    seq0 = s0_ref[0]
    seq10 = s10_ref[0]

    def sel(t):
        return mk1 * t[:, :H] + mk2 * t[:, H:2 * H] + mk3 * t[:, 2 * H:]

    hidden = jnp.zeros((B * V, H), jnp.float32)
    pred = None
    for s in range(NSTEP):
        ins = seq0 if s == 0 else (seq10 if s == PRED_STEPS else pred)
        hb = hidden.astype(bf)
        hr = jnp.dot(hb, w1t, preferred_element_type=jnp.float32).astype(bf)
        hs = jnp.dot(hb, w1b, preferred_element_type=jnp.float32).astype(bf)
        z = (jnp.dot(relrec_bf, hr, preferred_element_type=jnp.float32)
             + jnp.dot(relsend_bf, hs, preferred_element_type=jnp.float32)
             + b1c)
        a1 = jnp.tanh(sel(z))
        w = jnp.dot(a1.astype(bf), w2c,
                    preferred_element_type=jnp.float32) + b2c
        m = jnp.tanh(sel(w))
        agg = jnp.dot(relrt_bf, m.astype(bf),
                      preferred_element_type=jnp.float32) / 9.0
        gin = jnp.dot(ins, wi, preferred_element_type=jnp.float32) + bi
        gh = jnp.dot(agg, wh, preferred_element_type=jnp.float32)
        r = jax.nn.sigmoid(gin[:, :H] + gh[:, :H])
        i = jax.nn.sigmoid(gin[:, H:2 * H] + gh[:, H:2 * H])
        nn = jnp.tanh(gin[:, 2 * H:] + r * gh[:, 2 * H:])
        hidden = (1.0 - i) * nn + i * hidden
        p = jax.nn.relu(jnp.dot(hidden, o1w[...], preferred_element_type=jnp.float32) + o1b[...])
        p = jax.nn.relu(jnp.dot(p, o2w[...], preferred_element_type=jnp.float32) + o2b[...])
        p = jnp.dot(p, o3w[...], preferred_element_type=jnp.float32) + o3b[...]
        pred = ins + p
        preds_ref[0, :, s, :] = pred


def _row(b):
    return b.reshape(1, -1)


def kernel(data, params, rel_rec, rel_send):
    p = params
    bf = jnp.bfloat16
    x = jnp.transpose(data, (0, 3, 1, 2)).reshape(G, B * V, N_IN_ENC)
    seq = jnp.transpose(data, (0, 2, 3, 1))
    s0 = seq[:, 0].reshape(G, B * V, C)
    s10 = seq[:, PRED_STEPS].reshape(G, B * V, C)
    u = jax.random.uniform(jax.random.key(42), (N, E, K),
                           dtype=jnp.float32).reshape(G, B * E, K)
    eyeb = jnp.eye(B, dtype=jnp.float32)
    bigrec = jnp.kron(eyeb, rel_rec)
    bigsend = jnp.kron(eyeb, rel_send)
    relcat = jnp.concatenate([bigrec, bigsend], axis=0)
    relrt = jnp.kron(eyeb, jnp.transpose(rel_rec))

    weights = []
    for name in ('mlp1', 'mlp2', 'mlp3', 'mlp4'):
        weights += [p[name + '_fc1_w'], _row(p[name + '_fc1_b']),
                    p[name + '_fc2_w'], _row(p[name + '_fc2_b']),
                    jnp.stack([p[name + '_bn_g'], p[name + '_bn_b']])]
    weights += [p['fc_out_w'], _row(p['fc_out_b'])]
    weights += [jnp.concatenate([_row(p['msg1_%d_b' % k]) for k in (1, 2, 3)], axis=1),
                jnp.concatenate([_row(p['msg2_%d_b' % k]) for k in (1, 2, 3)], axis=1)]
    weights += [jnp.concatenate([p['input_r_w'], p['input_i_w'], p['input_n_w']], axis=1),
                jnp.concatenate([_row(p['input_r_b']), _row(p['input_i_b']),
                                 _row(p['input_n_b'])], axis=1),
                jnp.concatenate([p['hidden_r_w'], p['hidden_i_w'], p['hidden_n_w']], axis=1)]
    weights += [p['out_fc1_w'], _row(p['out_fc1_b']),
                p['out_fc2_w'], _row(p['out_fc2_b']),
                p['out_fc3_w'], _row(p['out_fc3_b'])]
    w1cat = jnp.concatenate([p['msg1_%d_w' % k] for k in (1, 2, 3)], axis=1)
    weights += [bigrec.astype(bf), bigsend.astype(bf), relrt.astype(bf),
                w1cat[:H].astype(bf), w1cat[H:].astype(bf),
                jnp.concatenate([p['msg2_%d_w' % k] for k in (1, 2, 3)],
                                axis=1).astype(bf)]

    def wspec(w):
        nd = w.ndim
        return pl.BlockSpec(w.shape, lambda i, _nd=nd: (0,) * _nd)

    in_specs = [
        pl.BlockSpec((1, B * V, N_IN_ENC), lambda i: (i, 0, 0)),
        pl.BlockSpec((1, B * V, C), lambda i: (i, 0, 0)),
        pl.BlockSpec((1, B * V, C), lambda i: (i, 0, 0)),
        pl.BlockSpec((1, B * E, K), lambda i: (i, 0, 0)),
        pl.BlockSpec((2 * B * E, B * V), lambda i: (0, 0)),
        pl.BlockSpec((B * V, B * E), lambda i: (0, 0)),
    ] + [wspec(w) for w in weights]

    out_specs = [
        pl.BlockSpec((1, B * V, NSTEP, C), lambda i: (i, 0, 0, 0)),
        pl.BlockSpec((1, B * E, K), lambda i: (i, 0, 0)),
    ]
    out_shape = [
        jax.ShapeDtypeStruct((G, B * V, NSTEP, C), jnp.float32),
        jax.ShapeDtypeStruct((G, B * E, K), jnp.float32),
    ]

    preds, prob = pl.pallas_call(
        _fwd_body,
        grid=(G,),
        in_specs=in_specs,
        out_specs=out_specs,
        out_shape=out_shape,
    )(x, s0, s10, u, relcat, relrt, *weights)
    return preds.reshape(N, V, NSTEP, C), prob.reshape(N, E, K)


# speedup vs baseline: 1.2601x; 1.2337x over previous
import jax
import jax.numpy as jnp
import numpy as np
from jax.experimental import pallas as pl

V = 25
E = V * (V - 1)
H = 128
K = 4
C = 3
T = 20
N = 32
N_IN_ENC = C * T
TAU = 0.5
PRED_STEPS = 10
NSTEP = T - 1
B = 4
G = N // B
BV = B * V
BE = B * E

_BN_DIV = float(np.sqrt(1.0 + 1e-5))

_PERM_I2O = np.empty((E,), np.int32)
_PERM_O2I = np.empty((E,), np.int32)
_i = 0
for _r in range(V):
    for _s in range(V):
        if _s == _r:
            continue
        _o = _s * (V - 1) + (_r if _r < _s else _r - 1)
        _PERM_I2O[_i] = _o
        _PERM_O2I[_o] = _i
        _i += 1


def _elu(x):
    return jnp.where(x > 0, x, jnp.exp(x) - 1.0)


def _bcast_recv(t):
    return jnp.broadcast_to(t[:, None, :], (BV, V - 1, t.shape[1])
                            ).reshape(BE, t.shape[1])


def _seg_sum(t):
    return jnp.sum(t.reshape(BV, V - 1, t.shape[1]), axis=1)


def _fwd_body(x_ref, s0_ref, s10_ref, u_ref, relsend_ref,
              m1w1, m1b1, m1w2, m1b2, m1bn,
              m2w1, m2b1, m2w2, m2b2, m2bn,
              m3w1, m3b1, m3w2, m3b2, m3bn,
              m4w1, m4b1, m4w2, m4b2, m4bn,
              fow, fob,
              mg1b, mg2b,
              gwi, gbi, gwh,
              o1w, o1b, o2w, o2b, o3w, o3b,
              relsend_bf_ref, w1top_bf, w1bot_bf, w2cat_bf,
              preds_ref, prob_ref):
    bf = jnp.bfloat16
    rsend = relsend_ref[...]

    x = x_ref[0]
    h = _elu(jnp.dot(x, m1w1[...], preferred_element_type=jnp.float32) + m1b1[...])
    h = _elu(jnp.dot(h, m1w2[...], preferred_element_type=jnp.float32) + m1b2[...])
    h = m1bn[0:1] * h / _BN_DIV + m1bn[1:2]

    w2a = m2w1[...]
    z = (_bcast_recv(jnp.dot(h, w2a[:H], preferred_element_type=jnp.float32))
         + jnp.dot(rsend, jnp.dot(h, w2a[H:], preferred_element_type=jnp.float32),
                   preferred_element_type=jnp.float32)
         + m2b1[...])
    h = _elu(z)
    h = _elu(jnp.dot(h, m2w2[...], preferred_element_type=jnp.float32) + m2b2[...])
    h = m2bn[0:1] * h / _BN_DIV + m2bn[1:2]
    h_skip = h

    inc = _seg_sum(h) / V
    h = _elu(jnp.dot(inc, m3w1[...], preferred_element_type=jnp.float32) + m3b1[...])
    h = _elu(jnp.dot(h, m3w2[...], preferred_element_type=jnp.float32) + m3b2[...])
    h = m3bn[0:1] * h / _BN_DIV + m3bn[1:2]

    w4a = m4w1[...]
    z = (_bcast_recv(jnp.dot(h, w4a[:H], preferred_element_type=jnp.float32))
         + jnp.dot(rsend, jnp.dot(h, w4a[H:2 * H], preferred_element_type=jnp.float32),
                   preferred_element_type=jnp.float32)
         + jnp.dot(h_skip, w4a[2 * H:], preferred_element_type=jnp.float32)
         + m4b1[...])
    h = _elu(z)
    h = _elu(jnp.dot(h, m4w2[...], preferred_element_type=jnp.float32) + m4b2[...])
    h = m4bn[0:1] * h / _BN_DIV + m4bn[1:2]
    logits = jnp.dot(h, fow[...], preferred_element_type=jnp.float32) + fob[...]

    prob_ref[0] = jax.nn.softmax(logits, axis=-1)

    u = u_ref[0]
    g = -jnp.log(1e-10 - jnp.log(u + 1e-10))
    y = (logits + g) / TAU
    y_soft = jax.nn.softmax(y, axis=-1)
    idx = jnp.zeros((BE, 1), jnp.int32)
    best = y[:, 0:1]
    for j in range(1, K):
        c = y[:, j:j + 1] > best
        idx = jnp.where(c, j, idx)
        best = jnp.where(c, y[:, j:j + 1], best)
    lane = jax.lax.broadcasted_iota(jnp.int32, (BE, K), 1)
    hard = (lane == idx).astype(jnp.float32)
    edges = (hard - y_soft) + y_soft
    mk1 = edges[:, 1:2]
    mk2 = edges[:, 2:3]
    mk3 = edges[:, 3:4]

    rsend_bf = relsend_bf_ref[...]
    w1t = w1top_bf[...]
    w1b = w1bot_bf[...]
    w2c = w2cat_bf[...]
    b1c = mg1b[...]
    b2c = mg2b[...]
    wi = gwi[...]; bi = gbi[...]; wh = gwh[...]
    seq0 = s0_ref[0]
    seq10 = s10_ref[0]

    def sel(t):
        return mk1 * t[:, :H] + mk2 * t[:, H:2 * H] + mk3 * t[:, 2 * H:]

    hidden = jnp.zeros((BV, H), jnp.float32)
    pred = None
    for s in range(NSTEP):
        ins = seq0 if s == 0 else (seq10 if s == PRED_STEPS else pred)
        hb = hidden.astype(bf)
        hr = jnp.dot(hb, w1t, preferred_element_type=jnp.float32)
        hs = jnp.dot(hb, w1b, preferred_element_type=jnp.float32).astype(bf)
        z = (_bcast_recv(hr)
             + jnp.dot(rsend_bf, hs, preferred_element_type=jnp.float32)
             + b1c)
        a1 = jnp.tanh(sel(z))
        w = jnp.dot(a1.astype(bf), w2c,
                    preferred_element_type=jnp.float32) + b2c
        m = jnp.tanh(sel(w))
        agg = _seg_sum(m) / 9.0
        gin = jnp.dot(ins, wi, preferred_element_type=jnp.float32) + bi
        gh = jnp.dot(agg, wh, preferred_element_type=jnp.float32)
        r = jax.nn.sigmoid(gin[:, :H] + gh[:, :H])
        i = jax.nn.sigmoid(gin[:, H:2 * H] + gh[:, H:2 * H])
        nn = jnp.tanh(gin[:, 2 * H:] + r * gh[:, 2 * H:])
        hidden = (1.0 - i) * nn + i * hidden
        p = jax.nn.relu(jnp.dot(hidden, o1w[...], preferred_element_type=jnp.float32) + o1b[...])
        p = jax.nn.relu(jnp.dot(p, o2w[...], preferred_element_type=jnp.float32) + o2b[...])
        p = jnp.dot(p, o3w[...], preferred_element_type=jnp.float32) + o3b[...]
        pred = ins + p
        preds_ref[0, :, s, :] = pred


def _row(b):
    return b.reshape(1, -1)


def kernel(data, params, rel_rec, rel_send):
    p = params
    bf = jnp.bfloat16
    perm_i2o = jnp.asarray(_PERM_I2O)
    perm_o2i = jnp.asarray(_PERM_O2I)
    x = jnp.transpose(data, (0, 3, 1, 2)).reshape(G, BV, N_IN_ENC)
    seq = jnp.transpose(data, (0, 2, 3, 1))
    s0 = seq[:, 0].reshape(G, BV, C)
    s10 = seq[:, PRED_STEPS].reshape(G, BV, C)
    u = jax.random.uniform(jax.random.key(42), (N, E, K), dtype=jnp.float32)
    u = u[:, perm_i2o, :].reshape(G, BE, K)
    eyeb = jnp.eye(B, dtype=jnp.float32)
    rsend_perm = rel_send[perm_i2o]
    bigsend = jnp.kron(eyeb, rsend_perm)

    weights = []
    for name in ('mlp1', 'mlp2', 'mlp3', 'mlp4'):
        weights += [p[name + '_fc1_w'], _row(p[name + '_fc1_b']),
                    p[name + '_fc2_w'], _row(p[name + '_fc2_b']),
                    jnp.stack([p[name + '_bn_g'], p[name + '_bn_b']])]
    weights += [p['fc_out_w'], _row(p['fc_out_b'])]
    weights += [jnp.concatenate([_row(p['msg1_%d_b' % k]) for k in (1, 2, 3)], axis=1),
                jnp.concatenate([_row(p['msg2_%d_b' % k]) for k in (1, 2, 3)], axis=1)]
    weights += [jnp.concatenate([p['input_r_w'], p['input_i_w'], p['input_n_w']], axis=1),
                jnp.concatenate([_row(p['input_r_b']), _row(p['input_i_b']),
                                 _row(p['input_n_b'])], axis=1),
                jnp.concatenate([p['hidden_r_w'], p['hidden_i_w'], p['hidden_n_w']], axis=1)]
    weights += [p['out_fc1_w'], _row(p['out_fc1_b']),
                p['out_fc2_w'], _row(p['out_fc2_b']),
                p['out_fc3_w'], _row(p['out_fc3_b'])]
    w1cat = jnp.concatenate([p['msg1_%d_w' % k] for k in (1, 2, 3)], axis=1)
    weights += [bigsend.astype(bf),
                w1cat[:H].astype(bf), w1cat[H:].astype(bf),
                jnp.concatenate([p['msg2_%d_w' % k] for k in (1, 2, 3)],
                                axis=1).astype(bf)]

    def wspec(w):
        nd = w.ndim
        return pl.BlockSpec(w.shape, lambda i, _nd=nd: (0,) * _nd)

    in_specs = [
        pl.BlockSpec((1, BV, N_IN_ENC), lambda i: (i, 0, 0)),
        pl.BlockSpec((1, BV, C), lambda i: (i, 0, 0)),
        pl.BlockSpec((1, BV, C), lambda i: (i, 0, 0)),
        pl.BlockSpec((1, BE, K), lambda i: (i, 0, 0)),
        pl.BlockSpec((BE, BV), lambda i: (0, 0)),
    ] + [wspec(w) for w in weights]

    out_specs = [
        pl.BlockSpec((1, BV, NSTEP, C), lambda i: (i, 0, 0, 0)),
        pl.BlockSpec((1, BE, K), lambda i: (i, 0, 0)),
    ]
    out_shape = [
        jax.ShapeDtypeStruct((G, BV, NSTEP, C), jnp.float32),
        jax.ShapeDtypeStruct((G, BE, K), jnp.float32),
    ]

    preds, prob = pl.pallas_call(
        _fwd_body,
        grid=(G,),
        in_specs=in_specs,
        out_specs=out_specs,
        out_shape=out_shape,
    )(x, s0, s10, u, bigsend, *weights)
    return (preds.reshape(N, V, NSTEP, C),
            prob.reshape(N, E, K)[:, perm_o2i, :])


# speedup vs baseline: 1.2647x; 1.0036x over previous
import jax
import jax.numpy as jnp
import numpy as np
from jax.experimental import pallas as pl

V = 25
E = V * (V - 1)
H = 128
K = 4
C = 3
T = 20
N = 32
N_IN_ENC = C * T
TAU = 0.5
PRED_STEPS = 10
NSTEP = T - 1
B = 4
G = N // B
BV = B * V
BE = B * E

_BN_DIV = float(np.sqrt(1.0 + 1e-5))

_PERM_I2O = np.empty((E,), np.int32)
_PERM_O2I = np.empty((E,), np.int32)
_i = 0
for _r in range(V):
    for _s in range(V):
        if _s == _r:
            continue
        _o = _s * (V - 1) + (_r if _r < _s else _r - 1)
        _PERM_I2O[_i] = _o
        _PERM_O2I[_o] = _i
        _i += 1


def _elu(x):
    return jnp.where(x > 0, x, jnp.exp(x) - 1.0)


def _bcast_recv(t):
    return jnp.broadcast_to(t[:, None, :], (BV, V - 1, t.shape[1])
                            ).reshape(BE, t.shape[1])


def _seg_sum(t):
    return jnp.sum(t.reshape(BV, V - 1, t.shape[1]), axis=1)


def _fwd_body(x_ref, s0_ref, s10_ref, u_ref, relsend_ref,
              m1w1, m1b1, m1w2, m1b2, m1bn,
              m2w1, m2b1, m2w2, m2b2, m2bn,
              m3w1, m3b1, m3w2, m3b2, m3bn,
              m4w1, m4b1, m4w2, m4b2, m4bn,
              fow, fob,
              mg1b, mg2b,
              gwi, gbi, gwh,
              o1w, o1b, o2w, o2b, o3w, o3b,
              relsend_bf_ref, w1top_bf, w1bot_bf, w2cat_bf,
              preds_ref, prob_ref):
    bf = jnp.bfloat16
    rsend = relsend_ref[...]

    x = x_ref[0]
    h = _elu(jnp.dot(x, m1w1[...], preferred_element_type=jnp.float32) + m1b1[...])
    h = _elu(jnp.dot(h, m1w2[...], preferred_element_type=jnp.float32) + m1b2[...])
    h = m1bn[0:1] * h / _BN_DIV + m1bn[1:2]

    w2a = m2w1[...]
    z = (_bcast_recv(jnp.dot(h, w2a[:H], preferred_element_type=jnp.float32))
         + jnp.dot(rsend, jnp.dot(h, w2a[H:], preferred_element_type=jnp.float32),
                   preferred_element_type=jnp.float32)
         + m2b1[...])
    h = _elu(z)
    h = _elu(jnp.dot(h, m2w2[...], preferred_element_type=jnp.float32) + m2b2[...])
    h = m2bn[0:1] * h / _BN_DIV + m2bn[1:2]
    h_skip = h

    inc = _seg_sum(h) / V
    h = _elu(jnp.dot(inc, m3w1[...], preferred_element_type=jnp.float32) + m3b1[...])
    h = _elu(jnp.dot(h, m3w2[...], preferred_element_type=jnp.float32) + m3b2[...])
    h = m3bn[0:1] * h / _BN_DIV + m3bn[1:2]

    w4a = m4w1[...]
    z = (_bcast_recv(jnp.dot(h, w4a[:H], preferred_element_type=jnp.float32))
         + jnp.dot(rsend, jnp.dot(h, w4a[H:2 * H], preferred_element_type=jnp.float32),
                   preferred_element_type=jnp.float32)
         + jnp.dot(h_skip, w4a[2 * H:], preferred_element_type=jnp.float32)
         + m4b1[...])
    h = _elu(z)
    h = _elu(jnp.dot(h, m4w2[...], preferred_element_type=jnp.float32) + m4b2[...])
    h = m4bn[0:1] * h / _BN_DIV + m4bn[1:2]
    logits = jnp.dot(h, fow[...], preferred_element_type=jnp.float32) + fob[...]

    prob_ref[0] = jax.nn.softmax(logits, axis=-1)

    u = u_ref[0]
    g = -jnp.log(1e-10 - jnp.log(u + 1e-10))
    y = (logits + g) / TAU
    y_soft = jax.nn.softmax(y, axis=-1)
    idx = jnp.zeros((BE, 1), jnp.int32)
    best = y[:, 0:1]
    for j in range(1, K):
        c = y[:, j:j + 1] > best
        idx = jnp.where(c, j, idx)
        best = jnp.where(c, y[:, j:j + 1], best)
    lane = jax.lax.broadcasted_iota(jnp.int32, (BE, K), 1)
    hard = (lane == idx).astype(jnp.float32)
    edges = (hard - y_soft) + y_soft
    mk1 = edges[:, 1:2]
    mk2 = edges[:, 2:3]
    mk3 = edges[:, 3:4]

    rsend_bf = relsend_bf_ref[...]
    w1t = w1top_bf[...]
    w1b = w1bot_bf[...]
    w2c = w2cat_bf[...]
    b1c = mg1b[...]
    b2c = mg2b[...]
    wi = gwi[...]; bi = gbi[...]; wh = gwh[...]
    seq0 = s0_ref[0]
    seq10 = s10_ref[0]

    def sel(t):
        return mk1 * t[:, :H] + mk2 * t[:, H:2 * H] + mk3 * t[:, 2 * H:]

    hidden = jnp.zeros((BV, H), jnp.float32)
    pred = None
    for s in range(NSTEP):
        ins = seq0 if s == 0 else (seq10 if s == PRED_STEPS else pred)
        hr = jnp.dot(hidden, w1t, preferred_element_type=jnp.float32)
        hs = jnp.dot(hidden, w1b, preferred_element_type=jnp.float32)
        z = (_bcast_recv(hr)
             + jnp.dot(rsend, hs, preferred_element_type=jnp.float32)
             + b1c)
        a1 = jnp.tanh(sel(z))
        w = jnp.dot(a1, w2c, preferred_element_type=jnp.float32) + b2c
        m = jnp.tanh(sel(w))
        agg = _seg_sum(m) / 9.0
        gin = jnp.dot(ins, wi, preferred_element_type=jnp.float32) + bi
        gh = jnp.dot(agg, wh, preferred_element_type=jnp.float32)
        r = jax.nn.sigmoid(gin[:, :H] + gh[:, :H])
        i = jax.nn.sigmoid(gin[:, H:2 * H] + gh[:, H:2 * H])
        nn = jnp.tanh(gin[:, 2 * H:] + r * gh[:, 2 * H:])
        hidden = (1.0 - i) * nn + i * hidden
        p = jax.nn.relu(jnp.dot(hidden, o1w[...], preferred_element_type=jnp.float32) + o1b[...])
        p = jax.nn.relu(jnp.dot(p, o2w[...], preferred_element_type=jnp.float32) + o2b[...])
        p = jnp.dot(p, o3w[...], preferred_element_type=jnp.float32) + o3b[...]
        pred = ins + p
        preds_ref[0, :, s, :] = pred


def _row(b):
    return b.reshape(1, -1)


def kernel(data, params, rel_rec, rel_send):
    p = params
    bf = jnp.bfloat16
    perm_i2o = jnp.asarray(_PERM_I2O)
    perm_o2i = jnp.asarray(_PERM_O2I)
    x = jnp.transpose(data, (0, 3, 1, 2)).reshape(G, BV, N_IN_ENC)
    seq = jnp.transpose(data, (0, 2, 3, 1))
    s0 = seq[:, 0].reshape(G, BV, C)
    s10 = seq[:, PRED_STEPS].reshape(G, BV, C)
    u = jax.random.uniform(jax.random.key(42), (N, E, K), dtype=jnp.float32)
    u = u[:, perm_i2o, :].reshape(G, BE, K)
    eyeb = jnp.eye(B, dtype=jnp.float32)
    bigsend = jnp.kron(eyeb, rel_send[perm_i2o])

    weights = []
    for name in ('mlp1', 'mlp2', 'mlp3', 'mlp4'):
        weights += [p[name + '_fc1_w'], _row(p[name + '_fc1_b']),
                    p[name + '_fc2_w'], _row(p[name + '_fc2_b']),
                    jnp.stack([p[name + '_bn_g'], p[name + '_bn_b']])]
    weights += [p['fc_out_w'], _row(p['fc_out_b'])]
    weights += [jnp.concatenate([_row(p['msg1_%d_b' % k]) for k in (1, 2, 3)], axis=1),
                jnp.concatenate([_row(p['msg2_%d_b' % k]) for k in (1, 2, 3)], axis=1)]
    weights += [jnp.concatenate([p['input_r_w'], p['input_i_w'], p['input_n_w']], axis=1),
                jnp.concatenate([_row(p['input_r_b']), _row(p['input_i_b']),
                                 _row(p['input_n_b'])], axis=1),
                jnp.concatenate([p['hidden_r_w'], p['hidden_i_w'], p['hidden_n_w']], axis=1)]
    weights += [p['out_fc1_w'], _row(p['out_fc1_b']),
                p['out_fc2_w'], _row(p['out_fc2_b']),
                p['out_fc3_w'], _row(p['out_fc3_b'])]
    w1cat = jnp.concatenate([p['msg1_%d_w' % k] for k in (1, 2, 3)], axis=1)
    weights += [bigsend,
                w1cat[:H], w1cat[H:],
                jnp.concatenate([p['msg2_%d_w' % k] for k in (1, 2, 3)],
                                axis=1)]

    def wspec(w):
        nd = w.ndim
        return pl.BlockSpec(w.shape, lambda i, _nd=nd: (0,) * _nd)

    in_specs = [
        pl.BlockSpec((1, BV, N_IN_ENC), lambda i: (i, 0, 0)),
        pl.BlockSpec((1, BV, C), lambda i: (i, 0, 0)),
        pl.BlockSpec((1, BV, C), lambda i: (i, 0, 0)),
        pl.BlockSpec((1, BE, K), lambda i: (i, 0, 0)),
        pl.BlockSpec((BE, BV), lambda i: (0, 0)),
    ] + [wspec(w) for w in weights]

    out_specs = [
        pl.BlockSpec((1, BV, NSTEP, C), lambda i: (i, 0, 0, 0)),
        pl.BlockSpec((1, BE, K), lambda i: (i, 0, 0)),
    ]
    out_shape = [
        jax.ShapeDtypeStruct((G, BV, NSTEP, C), jnp.float32),
        jax.ShapeDtypeStruct((G, BE, K), jnp.float32),
    ]

    preds, prob = pl.pallas_call(
        _fwd_body,
        grid=(G,),
        in_specs=in_specs,
        out_specs=out_specs,
        out_shape=out_shape,
    )(x, s0, s10, u, bigsend, *weights)
    return (preds.reshape(N, V, NSTEP, C),
            prob.reshape(N, E, K)[:, perm_o2i, :])


# speedup vs baseline: 1.6677x; 1.3187x over previous
import functools

import jax
import jax.numpy as jnp
import numpy as np
from jax import lax
from jax.experimental import pallas as pl
from jax.experimental.pallas import tpu as pltpu
from jax.experimental.pallas import tpu_sc as plsc

V = 25
E = V * (V - 1)
H = 128
K = 4
C = 3
T = 20
N = 32
NE = N * E
N_IN_ENC = C * T
TAU = 0.5
PRED_STEPS = 10
NSTEP = T - 1
B = 4
G = N // B
BV = B * V
BE = B * E

_BN_DIV = float(np.sqrt(1.0 + 1e-5))

_SCW = 30
_PW = NE // _SCW
_PV = _PW // 16

_PERM_I2O = np.empty((E,), np.int32)
_PERM_O2I = np.empty((E,), np.int32)
_i = 0
for _r in range(V):
    for _s in range(V):
        if _s == _r:
            continue
        _o = _s * (V - 1) + (_r if _r < _s else _r - 1)
        _PERM_I2O[_i] = _o
        _PERM_O2I[_o] = _i
        _i += 1


def _elu(x):
    return jnp.where(x > 0, x, jnp.exp(x) - 1.0)


def _bcast_recv(t):
    return jnp.broadcast_to(t[:, None, :], (BV, V - 1, t.shape[1])
                            ).reshape(BE, t.shape[1])


def _seg_sum(t):
    return jnp.sum(t.reshape(BV, V - 1, t.shape[1]), axis=1)


def _enc_body(x_ref, relsend_ref,
              m1w1, m1b1, m1w2, m1b2, m1bn,
              m2w1, m2b1, m2w2, m2b2, m2bn,
              m3w1, m3b1, m3w2, m3b2, m3bn,
              m4w1, m4b1, m4w2, m4b2, m4bn,
              fow, fob,
              logits_ref):
    rsend = relsend_ref[...]

    x = x_ref[0]
    h = _elu(jnp.dot(x, m1w1[...], preferred_element_type=jnp.float32) + m1b1[...])
    h = _elu(jnp.dot(h, m1w2[...], preferred_element_type=jnp.float32) + m1b2[...])
    h = m1bn[0:1] * h / _BN_DIV + m1bn[1:2]

    w2a = m2w1[...]
    z = (_bcast_recv(jnp.dot(h, w2a[:H], preferred_element_type=jnp.float32))
         + jnp.dot(rsend, jnp.dot(h, w2a[H:], preferred_element_type=jnp.float32),
                   preferred_element_type=jnp.float32)
         + m2b1[...])
    h = _elu(z)
    h = _elu(jnp.dot(h, m2w2[...], preferred_element_type=jnp.float32) + m2b2[...])
    h = m2bn[0:1] * h / _BN_DIV + m2bn[1:2]
    h_skip = h

    inc = _seg_sum(h) / V
    h = _elu(jnp.dot(inc, m3w1[...], preferred_element_type=jnp.float32) + m3b1[...])
    h = _elu(jnp.dot(h, m3w2[...], preferred_element_type=jnp.float32) + m3b2[...])
    h = m3bn[0:1] * h / _BN_DIV + m3bn[1:2]

    w4a = m4w1[...]
    z = (_bcast_recv(jnp.dot(h, w4a[:H], preferred_element_type=jnp.float32))
         + jnp.dot(rsend, jnp.dot(h, w4a[H:2 * H], preferred_element_type=jnp.float32),
                   preferred_element_type=jnp.float32)
         + jnp.dot(h_skip, w4a[2 * H:], preferred_element_type=jnp.float32)
         + m4b1[...])
    h = _elu(z)
    h = _elu(jnp.dot(h, m4w2[...], preferred_element_type=jnp.float32) + m4b2[...])
    h = m4bn[0:1] * h / _BN_DIV + m4bn[1:2]
    logits_ref[0] = jnp.dot(h, fow[...], preferred_element_type=jnp.float32) + fob[...]


def _sampler_body(l0, l1, l2, l3, g0, g1, g2, g3,
                  e1o, e2o, e3o, p0o, p1o, p2o, p3o,
                  lv0, lv1, lv2, lv3, gv0, gv1, gv2, gv3,
                  ev1, ev2, ev3, pv0, pv1, pv2, pv3):
    wid = lax.axis_index("s") * 2 + lax.axis_index("c")
    lvs = (lv0, lv1, lv2, lv3)
    gvs = (gv0, gv1, gv2, gv3)
    pvs = (pv0, pv1, pv2, pv3)

    @pl.when(wid < _SCW)
    def _():
        base = wid * _PW
        sl_h = pl.ds(base, _PW)
        for ref, vm in zip((l0, l1, l2, l3, g0, g1, g2, g3),
                           lvs + gvs):
            pltpu.sync_copy(ref.at[sl_h], vm)
        for i in range(_PV):
            sl = pl.ds(i * 16, 16)
            ls = [lvs[j][sl] for j in range(4)]
            ys = [ls[j] + gvs[j][sl] for j in range(4)]
            one = jnp.full((16,), 1.0, jnp.float32)
            zero = jnp.zeros((16,), jnp.float32)
            best1 = jnp.maximum(ys[0], ys[1])
            best2 = jnp.maximum(best1, ys[2])
            k1 = jnp.where(ys[1] > ys[0], one, zero)
            k2 = jnp.where(ys[2] > best1, one, zero)
            k3 = jnp.where(ys[3] > best2, one, zero)
            ev3[sl] = k3
            ev2[sl] = k2 * (1.0 - k3)
            ev1[sl] = k1 * (1.0 - k2) * (1.0 - k3)
            m = jnp.maximum(jnp.maximum(ls[0], ls[1]),
                            jnp.maximum(ls[2], ls[3]))
            xs = [jnp.exp(ls[j] - m) for j in range(4)]
            ssum = (xs[0] + xs[1]) + (xs[2] + xs[3])
            for j in range(4):
                pvs[j][sl] = xs[j] / ssum
        for vm, ref in zip((ev1, ev2, ev3) + pvs,
                           (e1o, e2o, e3o, p0o, p1o, p2o, p3o)):
            pltpu.sync_copy(vm, ref.at[sl_h])


def _run_sampler(lcols, gcols):
    mesh = plsc.VectorSubcoreMesh(core_axis_name="c", subcore_axis_name="s")
    f32 = jnp.float32
    out_t = [jax.ShapeDtypeStruct((NE,), f32)] * 7
    kern = functools.partial(
        pl.kernel, out_type=out_t, mesh=mesh,
        scratch_types=[pltpu.VMEM((_PW,), f32)] * 15,
    )(_sampler_body)
    return kern(lcols[0], lcols[1], lcols[2], lcols[3],
                gcols[0], gcols[1], gcols[2], gcols[3])


def _dec_body(s0_ref, s10_ref, edges_ref, relsend_ref,
              mg1b, mg2b,
              gwi, gbi, gwh,
              o1w, o1b, o2w, o2b, o3w, o3b,
              w1top, w1bot, w2cat,
              preds_ref):
    rsend = relsend_ref[...]
    w1t = w1top[...]
    w1b = w1bot[...]
    w2c = w2cat[...]
    b1c = mg1b[...]
    b2c = mg2b[...]
    wi = gwi[...]; bi = gbi[...]; wh = gwh[...]
    seq0 = s0_ref[0]
    seq10 = s10_ref[0]
    ed = edges_ref[0]
    mk1 = ed[:, 0:1]
    mk2 = ed[:, 1:2]
    mk3 = ed[:, 2:3]

    def sel(t):
        return mk1 * t[:, :H] + mk2 * t[:, H:2 * H] + mk3 * t[:, 2 * H:]

    hidden = jnp.zeros((BV, H), jnp.float32)
    pred = None
    for s in range(NSTEP):
        ins = seq0 if s == 0 else (seq10 if s == PRED_STEPS else pred)
        hr = jnp.dot(hidden, w1t, preferred_element_type=jnp.float32)
        hs = jnp.dot(hidden, w1b, preferred_element_type=jnp.float32)
        z = (_bcast_recv(hr)
             + jnp.dot(rsend, hs, preferred_element_type=jnp.float32)
             + b1c)
        a1 = jnp.tanh(sel(z))
        w = jnp.dot(a1, w2c, preferred_element_type=jnp.float32) + b2c
        m = jnp.tanh(sel(w))
        agg = _seg_sum(m) / 9.0
        gin = jnp.dot(ins, wi, preferred_element_type=jnp.float32) + bi
        gh = jnp.dot(agg, wh, preferred_element_type=jnp.float32)
        r = jax.nn.sigmoid(gin[:, :H] + gh[:, :H])
        i = jax.nn.sigmoid(gin[:, H:2 * H] + gh[:, H:2 * H])
        nn = jnp.tanh(gin[:, 2 * H:] + r * gh[:, 2 * H:])
        hidden = (1.0 - i) * nn + i * hidden
        p = jax.nn.relu(jnp.dot(hidden, o1w[...], preferred_element_type=jnp.float32) + o1b[...])
        p = jax.nn.relu(jnp.dot(p, o2w[...], preferred_element_type=jnp.float32) + o2b[...])
        p = jnp.dot(p, o3w[...], preferred_element_type=jnp.float32) + o3b[...]
        pred = ins + p
        preds_ref[0, :, s, :] = pred


def _row(b):
    return b.reshape(1, -1)


def _wspec(w):
    nd = w.ndim
    return pl.BlockSpec(w.shape, lambda i, _nd=nd: (0,) * _nd)


def kernel(data, params, rel_rec, rel_send):
    p = params
    perm_i2o = jnp.asarray(_PERM_I2O)
    perm_o2i = jnp.asarray(_PERM_O2I)
    x = jnp.transpose(data, (0, 3, 1, 2)).reshape(G, BV, N_IN_ENC)
    seq = jnp.transpose(data, (0, 2, 3, 1))
    s0 = seq[:, 0].reshape(G, BV, C)
    s10 = seq[:, PRED_STEPS].reshape(G, BV, C)
    u = jax.random.uniform(jax.random.key(42), (N, E, K), dtype=jnp.float32)
    u = u[:, perm_i2o, :]
    gum = -jnp.log(1e-10 - jnp.log(u + 1e-10))
    gcols = jnp.transpose(gum.reshape(NE, K))
    eyeb = jnp.eye(B, dtype=jnp.float32)
    bigsend = jnp.kron(eyeb, rel_send[perm_i2o])

    enc_w = []
    for name in ('mlp1', 'mlp2', 'mlp3', 'mlp4'):
        enc_w += [p[name + '_fc1_w'], _row(p[name + '_fc1_b']),
                  p[name + '_fc2_w'], _row(p[name + '_fc2_b']),
                  jnp.stack([p[name + '_bn_g'], p[name + '_bn_b']])]
    enc_w += [p['fc_out_w'], _row(p['fc_out_b'])]

    logits = pl.pallas_call(
        _enc_body,
        grid=(G,),
        in_specs=[pl.BlockSpec((1, BV, N_IN_ENC), lambda i: (i, 0, 0)),
                  pl.BlockSpec((BE, BV), lambda i: (0, 0))]
                 + [_wspec(w) for w in enc_w],
        out_specs=[pl.BlockSpec((1, BE, K), lambda i: (i, 0, 0))],
        out_shape=[jax.ShapeDtypeStruct((G, BE, K), jnp.float32)],
    )(x, bigsend, *enc_w)[0]

    lcols = jnp.transpose(logits.reshape(NE, K))
    e1, e2, e3, p0, p1, p2, p3 = _run_sampler(lcols, gcols)
    edges = jnp.stack([e1, e2, e3], axis=1).reshape(G, BE, 3)
    prob = jnp.stack([p0, p1, p2, p3], axis=1)

    dec_w = [jnp.concatenate([_row(p['msg1_%d_b' % k]) for k in (1, 2, 3)], axis=1),
             jnp.concatenate([_row(p['msg2_%d_b' % k]) for k in (1, 2, 3)], axis=1),
             jnp.concatenate([p['input_r_w'], p['input_i_w'], p['input_n_w']], axis=1),
             jnp.concatenate([_row(p['input_r_b']), _row(p['input_i_b']),
                              _row(p['input_n_b'])], axis=1),
             jnp.concatenate([p['hidden_r_w'], p['hidden_i_w'], p['hidden_n_w']], axis=1),
             p['out_fc1_w'], _row(p['out_fc1_b']),
             p['out_fc2_w'], _row(p['out_fc2_b']),
             p['out_fc3_w'], _row(p['out_fc3_b'])]
    w1cat = jnp.concatenate([p['msg1_%d_w' % k] for k in (1, 2, 3)], axis=1)
    dec_w += [w1cat[:H], w1cat[H:],
              jnp.concatenate([p['msg2_%d_w' % k] for k in (1, 2, 3)], axis=1)]

    preds = pl.pallas_call(
        _dec_body,
        grid=(G,),
        in_specs=[pl.BlockSpec((1, BV, C), lambda i: (i, 0, 0)),
                  pl.BlockSpec((1, BV, C), lambda i: (i, 0, 0)),
                  pl.BlockSpec((1, BE, 3), lambda i: (i, 0, 0)),
                  pl.BlockSpec((BE, BV), lambda i: (0, 0))]
                 + [_wspec(w) for w in dec_w],
        out_specs=[pl.BlockSpec((1, BV, NSTEP, C), lambda i: (i, 0, 0, 0))],
        out_shape=[jax.ShapeDtypeStruct((G, BV, NSTEP, C), jnp.float32)],
    )(s0, s10, edges, bigsend, *dec_w)[0]

    return (preds.reshape(N, V, NSTEP, C),
            prob.reshape(N, E, K)[:, perm_o2i, :])
